# Initial kernel scaffold; baseline (speedup 1.0000x reference)
import jax
import jax.numpy as jnp
from jax.experimental import pallas as pl


def kernel(original_ids, batch, emb_table, W1, b1, W2, b2):
    raise NotImplementedError("write your pallas kernel here")


# speedup vs baseline: 9.0888x; 9.0888x over previous
import functools

import jax
import jax.numpy as jnp
from jax import lax
from jax.experimental import pallas as pl
from jax.experimental.pallas import tpu as pltpu
from jax.experimental.pallas import tpu_sc as plsc

NUM_EMB = 105220
D = 256
G = 128
L = 50
N = 6400
H = 128

NC, NS = 2, 16
NW = NC * NS
CHUNK = 40


def _sc_gather(table, idx, d):
    n_rows = idx.shape[0]
    rpw = n_rows // NW
    n_chunks = rpw // CHUNK
    mesh = plsc.VectorSubcoreMesh(core_axis_name="c", subcore_axis_name="s")

    @functools.partial(
        pl.kernel,
        out_type=jax.ShapeDtypeStruct((n_rows, d), jnp.float32),
        mesh=mesh,
        scratch_types=[
            pltpu.VMEM((rpw,), jnp.int32),
            pltpu.VMEM((rpw, d), jnp.float32),
            pltpu.SemaphoreType.DMA,
        ],
    )
    def gather_kernel(idx_hbm, table_hbm, out_hbm, idx_v, rows_v, sem):
        wid = lax.axis_index("s") * NC + lax.axis_index("c")
        base = wid * rpw
        pltpu.sync_copy(idx_hbm.at[pl.ds(base, rpw)], idx_v)
        copies = [
            pltpu.async_copy(
                table_hbm.at[idx_v.at[pl.ds(j * CHUNK, CHUNK)]],
                rows_v.at[pl.ds(j * CHUNK, CHUNK)],
                sem,
            )
            for j in range(n_chunks)
        ]
        for c in copies:
            c.wait()
        pltpu.sync_copy(rows_v, out_hbm.at[pl.ds(base, rpw)])

    return gather_kernel(idx, table)


def _project_body(e_ref, w1_ref, b1_ref, w2_ref, b2_ref, m_ref):
    e = e_ref[...]
    h = jnp.dot(e, w1_ref[...], preferred_element_type=jnp.float32)
    h = h + b1_ref[...]
    h = jnp.where(h >= 0, h, 0.01 * h)
    row = lax.broadcasted_iota(jnp.int32, (G, G * L), 0)
    col = lax.broadcasted_iota(jnp.int32, (G, G * L), 1)
    off = col - row * L
    seg = jnp.where((off >= 0) & (off < L), 1.0 / L, 0.0)
    hm = jnp.dot(seg, h, preferred_element_type=jnp.float32)
    m = jnp.dot(hm, w2_ref[...], preferred_element_type=jnp.float32)
    m_ref[...] = m + b2_ref[...]


def _project(e, W1, b1, W2, b2):
    return pl.pallas_call(
        _project_body,
        out_shape=jax.ShapeDtypeStruct((G, D), jnp.float32),
    )(e, W1, b1.reshape(1, H), W2, b2.reshape(1, D))


def kernel(original_ids, batch, emb_table, W1, b1, W2, b2):
    ids = jnp.clip(original_ids.astype(jnp.int32), 0, NUM_EMB - 1)
    e = _sc_gather(emb_table, ids.reshape(-1), D)
    m = _project(e, W1, b1, W2, b2)
    return _sc_gather(m, batch.astype(jnp.int32), D)


# speedup vs baseline: 11.8626x; 1.3052x over previous
import functools

import jax
import jax.numpy as jnp
from jax import lax
from jax.experimental import pallas as pl
from jax.experimental.pallas import tpu as pltpu
from jax.experimental.pallas import tpu_sc as plsc

NUM_EMB = 105220
D = 256
G = 128
L = 50
N = 6400
H = 128

NC, NS = 2, 16
NW = NC * NS
CHUNK = 40


def _sc_gather(table, idx, d):
    n_rows = idx.shape[0]
    rpw = n_rows // NW
    n_chunks = rpw // CHUNK
    mesh = plsc.VectorSubcoreMesh(core_axis_name="c", subcore_axis_name="s")

    @functools.partial(
        pl.kernel,
        out_type=jax.ShapeDtypeStruct((n_rows, d), jnp.float32),
        mesh=mesh,
        scratch_types=[
            pltpu.VMEM((rpw,), jnp.int32),
            pltpu.VMEM((rpw, d), jnp.float32),
            pltpu.SemaphoreType.DMA,
        ],
    )
    def gather_kernel(idx_hbm, table_hbm, out_hbm, idx_v, rows_v, sem):
        wid = lax.axis_index("s") * NC + lax.axis_index("c")
        base = wid * rpw
        pltpu.sync_copy(idx_hbm.at[pl.ds(base, rpw)], idx_v)
        copies = [
            pltpu.async_copy(
                table_hbm.at[idx_v.at[pl.ds(j * CHUNK, CHUNK)]],
                rows_v.at[pl.ds(j * CHUNK, CHUNK)],
                sem,
            )
            for j in range(n_chunks)
        ]
        for c in copies:
            c.wait()
        pltpu.sync_copy(rows_v, out_hbm.at[pl.ds(base, rpw)])

    return gather_kernel(idx, table)


def _project_body(e_ref, w1_ref, b1_ref, w2_ref, b2_ref, batch_ref, x_ref):
    e = e_ref[...]
    h = jnp.dot(e, w1_ref[...], preferred_element_type=jnp.float32)
    h = h + b1_ref[...]
    h = jnp.where(h >= 0, h, 0.01 * h)
    row = lax.broadcasted_iota(jnp.int32, (G, G * L), 0)
    col = lax.broadcasted_iota(jnp.int32, (G, G * L), 1)
    off = col - row * L
    seg = jnp.where((off >= 0) & (off < L), 1.0 / L, 0.0)
    hm = jnp.dot(seg, h, preferred_element_type=jnp.float32)
    m = jnp.dot(hm, w2_ref[...], preferred_element_type=jnp.float32)
    m = m + b2_ref[...]
    gid = lax.broadcasted_iota(jnp.int32, (N, G), 1)
    onehot = jnp.where(batch_ref[...] == gid, 1.0, 0.0)
    x_ref[...] = jnp.dot(onehot, m, preferred_element_type=jnp.float32)


def _project(e, W1, b1, W2, b2, batch):
    return pl.pallas_call(
        _project_body,
        out_shape=jax.ShapeDtypeStruct((N, D), jnp.float32),
    )(e, W1, b1.reshape(1, H), W2, b2.reshape(1, D), batch.reshape(N, 1))


def kernel(original_ids, batch, emb_table, W1, b1, W2, b2):
    ids = jnp.clip(original_ids.astype(jnp.int32), 0, NUM_EMB - 1)
    e = _sc_gather(emb_table, ids.reshape(-1), D)
    return _project(e, W1, b1, W2, b2, batch.astype(jnp.int32))
